# baseline (speedup 1.0000x reference)
import functools

import jax
import jax.numpy as jnp
from jax import lax
from jax.experimental import pallas as pl
from jax.experimental.pallas import tpu as pltpu
from jax.experimental.pallas import tpu_sc as plsc

D = 768
E = 16
H = 1152
N = 4096
T = 256
MAXTILES = N // T + E
PAD = MAXTILES * T
RB = 512
NRB = N // RB


def _router_body(x_ref, wr_ref, eidx_ref, rank_ref, counts_ref, run_ref):
    b = pl.program_id(0)

    @pl.when(b == 0)
    def _():
        run_ref[...] = jnp.zeros_like(run_ref)

    xb = x_ref[...]
    wr = wr_ref[...]
    logits = lax.dot_general(xb, wr, (((1,), (1,)), ((), ())),
                             preferred_element_type=jnp.float32)
    lane = lax.broadcasted_iota(jnp.int32, (RB, E), 1)
    mx = jnp.max(logits, axis=1, keepdims=True)
    e = jnp.min(jnp.where(logits >= mx, lane, E), axis=1)
    oh = (e[:, None] == lane).astype(jnp.float32)

    row_i = lax.broadcasted_iota(jnp.int32, (RB, RB), 0)
    col_i = lax.broadcasted_iota(jnp.int32, (RB, RB), 1)
    lt = (col_i < row_i).astype(jnp.float32)
    rank_in = lax.dot_general(lt, oh, (((1,), (0,)), ((), ())),
                              preferred_element_type=jnp.float32)

    run = run_ref[...]
    run_b = jnp.broadcast_to(run[0:1], (RB, E))
    rank = jnp.sum((rank_in + run_b) * oh, axis=1)

    eidx_ref[...] = e.astype(jnp.int32).reshape(1, 1, RB)
    rank_ref[...] = rank.astype(jnp.int32).reshape(1, 1, RB)

    colsum = jnp.sum(oh, axis=0, keepdims=True)
    new_run = run + jnp.broadcast_to(colsum, (8, E))
    run_ref[...] = new_run
    counts_ref[...] = new_run


def _router(x_flat, Wr):
    eidx, rank, counts = pl.pallas_call(
        _router_body,
        grid=(NRB,),
        in_specs=[
            pl.BlockSpec((RB, D), lambda b: (b, 0)),
            pl.BlockSpec((E, D), lambda b: (0, 0)),
        ],
        out_specs=[
            pl.BlockSpec((1, 1, RB), lambda b: (b, 0, 0)),
            pl.BlockSpec((1, 1, RB), lambda b: (b, 0, 0)),
            pl.BlockSpec((8, E), lambda b: (0, 0)),
        ],
        out_shape=[
            jax.ShapeDtypeStruct((NRB, 1, RB), jnp.int32),
            jax.ShapeDtypeStruct((NRB, 1, RB), jnp.int32),
            jax.ShapeDtypeStruct((8, E), jnp.float32),
        ],
        scratch_shapes=[pltpu.VMEM((8, E), jnp.float32)],
    )(x_flat, Wr)
    return eidx.reshape(N), rank.reshape(N), counts[0].astype(jnp.int32)


@functools.lru_cache(maxsize=None)
def _make_sc_gather(n_rows, table_rows, chunk):
    nc, ns = 2, 16
    nw = nc * ns
    per_w = n_rows // nw
    n_chunks = per_w // chunk
    mesh = plsc.VectorSubcoreMesh(core_axis_name="c", subcore_axis_name="s",
                                  num_cores=nc, num_subcores=ns)

    @functools.partial(
        pl.kernel, mesh=mesh,
        out_type=jax.ShapeDtypeStruct((n_rows, D), jnp.float32),
        scratch_types=[
            pltpu.VMEM((chunk,), jnp.int32),
            pltpu.VMEM((chunk, D), jnp.float32),
            pltpu.SemaphoreType.DMA,
        ],
    )
    def k(table_hbm, idx_hbm, out_hbm, idx_v, rows_v, sem):
        wid = lax.axis_index("s") * nc + lax.axis_index("c")
        for c in range(n_chunks):
            base = wid * per_w + c * chunk
            pltpu.sync_copy(idx_hbm.at[pl.ds(base, chunk)], idx_v)
            pltpu.async_copy(table_hbm.at[idx_v], rows_v, sem).wait()
            pltpu.sync_copy(rows_v, out_hbm.at[pl.ds(base, chunk)])

    return k


def _dispatch(x_flat, row_to_token):
    return _make_sc_gather(PAD, N, 128)(x_flat, row_to_token)


def _combine(ys, slot):
    return _make_sc_gather(N, PAD, 128)(ys, slot)


NJ = 3
H2 = H // NJ


def _ffn_body(te_ref, tv_ref, xs_ref, wg_ref, wu_ref, wd_ref, out_ref):
    i = pl.program_id(0)
    j = pl.program_id(1)

    @pl.when(tv_ref[i] == 1)
    def _():
        xb = xs_ref[...]
        wg = wg_ref[0]
        wu = wu_ref[0]
        wd = wd_ref[0]
        g = lax.dot_general(xb, wg, (((1,), (1,)), ((), ())),
                            preferred_element_type=jnp.float32)
        u = lax.dot_general(xb, wu, (((1,), (1,)), ((), ())),
                            preferred_element_type=jnp.float32)
        g = g * (1.0 / (1.0 + jnp.exp(-g)))
        h = g * u
        part = lax.dot_general(h, wd, (((1,), (1,)), ((), ())),
                               preferred_element_type=jnp.float32)

        @pl.when(j == 0)
        def _():
            out_ref[...] = part

        @pl.when(j != 0)
        def _():
            out_ref[...] += part


def _ffn(tile_expert, tile_valid, xs, Wg, Wu, Wd):
    return pl.pallas_call(
        _ffn_body,
        grid_spec=pltpu.PrefetchScalarGridSpec(
            num_scalar_prefetch=2,
            grid=(MAXTILES, NJ),
            in_specs=[
                pl.BlockSpec((T, D), lambda i, j, te, tv: (i, 0)),
                pl.BlockSpec((1, H2, D), lambda i, j, te, tv: (te[i], j, 0)),
                pl.BlockSpec((1, H2, D), lambda i, j, te, tv: (te[i], j, 0)),
                pl.BlockSpec((1, D, H2), lambda i, j, te, tv: (te[i], 0, j)),
            ],
            out_specs=pl.BlockSpec((T, D), lambda i, j, te, tv: (i, 0)),
        ),
        out_shape=jax.ShapeDtypeStruct((PAD, D), jnp.float32),
    )(tile_expert, tile_valid, xs, Wg, Wu, Wd)


def kernel(x, Wr, Wg, Wu, Wd):
    B, L, _ = x.shape
    x_flat = x.reshape(N, D)

    eidx, rank, counts = _router(x_flat, Wr)

    nt_e = (counts + T - 1) // T
    csum_nt = jnp.cumsum(nt_e)
    tile_base = csum_nt - nt_e
    padded_base = tile_base * T
    slot = padded_base[eidx] + rank
    pad_fill = (jnp.arange(PAD, dtype=jnp.int32) * 1021) % N
    row_to_token = pad_fill.at[slot].set(jnp.arange(N, dtype=jnp.int32))
    tiles = jnp.arange(MAXTILES, dtype=jnp.int32)
    tile_expert = jnp.minimum(
        jnp.searchsorted(csum_nt, tiles, side="right"), E - 1).astype(jnp.int32)
    tile_valid = (tiles < csum_nt[-1]).astype(jnp.int32)

    xs = _dispatch(x_flat, row_to_token)
    ys = _ffn(tile_expert, tile_valid, xs, Wg, Wu, Wd)
    out_flat = _combine(ys, slot.astype(jnp.int32))
    return out_flat.reshape(B, L, D)


# speedup vs baseline: 1.3385x; 1.3385x over previous
import functools

import jax
import jax.numpy as jnp
from jax import lax
from jax.experimental import pallas as pl
from jax.experimental.pallas import tpu as pltpu
from jax.experimental.pallas import tpu_sc as plsc

D = 768
E = 16
H = 1152
N = 4096
T = 256
MAXTILES = N // T + E
PAD = MAXTILES * T
RB = 512
NRB = N // RB


def _router_body(x_ref, wr_ref, eidx_ref, rank_ref, counts_ref, run_ref):
    b = pl.program_id(0)

    @pl.when(b == 0)
    def _():
        run_ref[...] = jnp.zeros_like(run_ref)

    xb = x_ref[...]
    wr = wr_ref[...]
    logits = lax.dot_general(xb, wr, (((1,), (1,)), ((), ())),
                             preferred_element_type=jnp.float32)
    lane = lax.broadcasted_iota(jnp.int32, (RB, E), 1)
    mx = jnp.max(logits, axis=1, keepdims=True)
    e = jnp.min(jnp.where(logits >= mx, lane, E), axis=1)
    oh = (e[:, None] == lane).astype(jnp.float32)

    row_i = lax.broadcasted_iota(jnp.int32, (RB, RB), 0)
    col_i = lax.broadcasted_iota(jnp.int32, (RB, RB), 1)
    lt = (col_i < row_i).astype(jnp.float32)
    rank_in = lax.dot_general(lt, oh, (((1,), (0,)), ((), ())),
                              preferred_element_type=jnp.float32)

    run = run_ref[...]
    run_b = jnp.broadcast_to(run[0:1], (RB, E))
    rank = jnp.sum((rank_in + run_b) * oh, axis=1)

    eidx_ref[...] = e.astype(jnp.int32).reshape(1, 1, RB)
    rank_ref[...] = rank.astype(jnp.int32).reshape(1, 1, RB)

    colsum = jnp.sum(oh, axis=0, keepdims=True)
    new_run = run + jnp.broadcast_to(colsum, (8, E))
    run_ref[...] = new_run
    counts_ref[...] = new_run


def _router(x_flat, Wr):
    eidx, rank, counts = pl.pallas_call(
        _router_body,
        grid=(NRB,),
        in_specs=[
            pl.BlockSpec((RB, D), lambda b: (b, 0)),
            pl.BlockSpec((E, D), lambda b: (0, 0)),
        ],
        out_specs=[
            pl.BlockSpec((1, 1, RB), lambda b: (b, 0, 0)),
            pl.BlockSpec((1, 1, RB), lambda b: (b, 0, 0)),
            pl.BlockSpec((8, E), lambda b: (0, 0)),
        ],
        out_shape=[
            jax.ShapeDtypeStruct((NRB, 1, RB), jnp.int32),
            jax.ShapeDtypeStruct((NRB, 1, RB), jnp.int32),
            jax.ShapeDtypeStruct((8, E), jnp.float32),
        ],
        scratch_shapes=[pltpu.VMEM((8, E), jnp.float32)],
    )(x_flat, Wr)
    return eidx.reshape(N), rank.reshape(N), counts[0].astype(jnp.int32)


@functools.lru_cache(maxsize=None)
def _make_sc_gather(n_rows, table_rows, chunk):
    nc, ns = 2, 16
    nw = nc * ns
    per_w = n_rows // nw
    n_chunks = per_w // chunk
    mesh = plsc.VectorSubcoreMesh(core_axis_name="c", subcore_axis_name="s",
                                  num_cores=nc, num_subcores=ns)

    @functools.partial(
        pl.kernel, mesh=mesh,
        out_type=jax.ShapeDtypeStruct((n_rows, D), jnp.float32),
        scratch_types=[
            pltpu.VMEM((chunk,), jnp.int32),
            pltpu.VMEM((chunk, D), jnp.float32),
            pltpu.SemaphoreType.DMA,
        ],
    )
    def k(table_hbm, idx_hbm, out_hbm, idx_v, rows_v, sem):
        wid = lax.axis_index("s") * nc + lax.axis_index("c")
        for c in range(n_chunks):
            base = wid * per_w + c * chunk
            pltpu.sync_copy(idx_hbm.at[pl.ds(base, chunk)], idx_v)
            pltpu.async_copy(table_hbm.at[idx_v], rows_v, sem).wait()
            pltpu.sync_copy(rows_v, out_hbm.at[pl.ds(base, chunk)])

    return k


def _dispatch(x_flat, row_to_token):
    return _make_sc_gather(PAD, N, 128)(x_flat, row_to_token)


def _combine(ys, slot):
    return _make_sc_gather(N, PAD, 128)(ys, slot)


def _ffn_body(te_ref, tv_ref, xs_ref, wg_ref, wu_ref, wd_ref, out_ref):
    i = pl.program_id(0)

    @pl.when(tv_ref[i] == 1)
    def _():
        xb = xs_ref[...]
        wg = wg_ref[0]
        wu = wu_ref[0]
        wd = wd_ref[0]
        g = lax.dot_general(xb, wg, (((1,), (1,)), ((), ())),
                            preferred_element_type=jnp.float32)
        u = lax.dot_general(xb, wu, (((1,), (1,)), ((), ())),
                            preferred_element_type=jnp.float32)
        g = g * (1.0 / (1.0 + jnp.exp(-g)))
        h = g * u
        out_ref[...] = lax.dot_general(h, wd, (((1,), (1,)), ((), ())),
                                       preferred_element_type=jnp.float32)


def _ffn(tile_expert, tile_valid, xs, Wg, Wu, Wd):
    return pl.pallas_call(
        _ffn_body,
        grid_spec=pltpu.PrefetchScalarGridSpec(
            num_scalar_prefetch=2,
            grid=(MAXTILES,),
            in_specs=[
                pl.BlockSpec((T, D), lambda i, te, tv: (i, 0)),
                pl.BlockSpec((1, H, D), lambda i, te, tv: (te[i], 0, 0)),
                pl.BlockSpec((1, H, D), lambda i, te, tv: (te[i], 0, 0)),
                pl.BlockSpec((1, D, H), lambda i, te, tv: (te[i], 0, 0)),
            ],
            out_specs=pl.BlockSpec((T, D), lambda i, te, tv: (i, 0)),
        ),
        out_shape=jax.ShapeDtypeStruct((PAD, D), jnp.float32),
    )(tile_expert, tile_valid, xs, Wg, Wu, Wd)


def kernel(x, Wr, Wg, Wu, Wd):
    B, L, _ = x.shape
    x_flat = x.reshape(N, D)

    eidx, rank, counts = _router(x_flat, Wr)

    nt_e = (counts + T - 1) // T
    csum_nt = jnp.cumsum(nt_e)
    tile_base = csum_nt - nt_e
    padded_base = tile_base * T
    slot = padded_base[eidx] + rank
    pad_fill = (jnp.arange(PAD, dtype=jnp.int32) * 1021) % N
    row_to_token = pad_fill.at[slot].set(jnp.arange(N, dtype=jnp.int32))
    tiles = jnp.arange(MAXTILES, dtype=jnp.int32)
    tile_expert = jnp.minimum(
        jnp.searchsorted(csum_nt, tiles, side="right"), E - 1).astype(jnp.int32)
    tile_valid = (tiles < csum_nt[-1]).astype(jnp.int32)

    xs = _dispatch(x_flat, row_to_token)
    ys = _ffn(tile_expert, tile_valid, xs, Wg, Wu, Wd)
    out_flat = _combine(ys, slot.astype(jnp.int32))
    return out_flat.reshape(B, L, D)
